# Initial kernel scaffold; baseline (speedup 1.0000x reference)
import jax
import jax.numpy as jnp
from jax.experimental import pallas as pl


def kernel(input, table):
    raise NotImplementedError("write your pallas kernel here")


# speedup vs baseline: 3.8443x; 3.8443x over previous
import functools

import jax
import jax.numpy as jnp
from jax import lax
from jax.experimental import pallas as pl
from jax.experimental.pallas import tpu as pltpu
from jax.experimental.pallas import tpu_sc as plsc

SEQ_LEN = 200
NUM_HID = 128
BATCH = 4096

_info = plsc.get_sparse_core_info()
NC, NS, L = _info.num_cores, _info.num_subcores, _info.num_lanes
NW = NC * NS
SEQ_PER_W = BATCH // NW


def _pos_encode(seq_len, dim):
    range_even = jnp.arange(dim, dtype=jnp.float32)
    range_even = (range_even / 2.0).astype(jnp.int32).astype(jnp.float32)
    power = range_even / float(dim)
    denom = jnp.power(10000.0, power).reshape(1, dim)
    pos = jnp.arange(seq_len, dtype=jnp.float32).reshape(seq_len, 1)
    arg = pos / denom
    cos_mask = (jnp.arange(seq_len) % 2).astype(bool).reshape(seq_len, 1)
    sin = jnp.where(jnp.logical_not(cos_mask), jnp.sin(arg), 0.0)
    cos = jnp.where(cos_mask, jnp.cos(arg), 0.0)
    return sin + cos


def _sc_body(idx_hbm, table_hbm, pe_hbm, out_hbm, idx_v, rows_v, pe_v, sem):
    c = lax.axis_index("c")
    s = lax.axis_index("s")
    wid = s * NC + c

    pltpu.sync_copy(pe_hbm, pe_v)

    def seq_body(i, carry):
        b = wid * SEQ_PER_W + i
        pltpu.sync_copy(idx_hbm.at[b], idx_v)
        pltpu.async_copy(
            table_hbm.at[idx_v.at[pl.ds(0, 128)]], rows_v.at[pl.ds(0, 128)], sem
        ).wait()
        pltpu.async_copy(
            table_hbm.at[idx_v.at[pl.ds(128, 72)]], rows_v.at[pl.ds(128, 72)], sem
        ).wait()

        def row_body(r, carry2):
            for j in range(NUM_HID // L):
                sl = pl.ds(j * L, L)
                rows_v[r, sl] = rows_v[r, sl] + pe_v[r, sl]
            return carry2

        lax.fori_loop(0, SEQ_LEN, row_body, 0)
        pltpu.sync_copy(rows_v, out_hbm.at[b])
        return carry

    lax.fori_loop(0, SEQ_PER_W, seq_body, 0)


@jax.jit
def kernel(input, table):
    pe = _pos_encode(SEQ_LEN, NUM_HID)
    mesh = plsc.VectorSubcoreMesh(core_axis_name="c", subcore_axis_name="s")
    f = pl.kernel(
        _sc_body,
        out_type=jax.ShapeDtypeStruct((BATCH, SEQ_LEN, NUM_HID), jnp.float32),
        mesh=mesh,
        scratch_types=[
            pltpu.VMEM((SEQ_LEN,), jnp.int32),
            pltpu.VMEM((SEQ_LEN, NUM_HID), jnp.float32),
            pltpu.VMEM((SEQ_LEN, NUM_HID), jnp.float32),
            pltpu.SemaphoreType.DMA,
        ],
    )
    return f(input.astype(jnp.int32), table, pe)


# speedup vs baseline: 7.5809x; 1.9720x over previous
import functools

import jax
import jax.numpy as jnp
from jax import lax
from jax.experimental import pallas as pl
from jax.experimental.pallas import tpu as pltpu
from jax.experimental.pallas import tpu_sc as plsc

SEQ_LEN = 200
NUM_HID = 128
BATCH = 4096

_info = plsc.get_sparse_core_info()
NC, NS, L = _info.num_cores, _info.num_subcores, _info.num_lanes
NW = NC * NS
SEQ_PER_W = BATCH // NW


def _pos_encode(seq_len, dim):
    range_even = jnp.arange(dim, dtype=jnp.float32)
    range_even = (range_even / 2.0).astype(jnp.int32).astype(jnp.float32)
    power = range_even / float(dim)
    denom = jnp.power(10000.0, power).reshape(1, dim)
    pos = jnp.arange(seq_len, dtype=jnp.float32).reshape(seq_len, 1)
    arg = pos / denom
    cos_mask = (jnp.arange(seq_len) % 2).astype(bool).reshape(seq_len, 1)
    sin = jnp.where(jnp.logical_not(cos_mask), jnp.sin(arg), 0.0)
    cos = jnp.where(cos_mask, jnp.cos(arg), 0.0)
    return sin + cos


def _sc_body(
    idx_hbm, table_hbm, pe_hbm, out_hbm, idx_v, rows0, rows1, pe_v, sem0, sem1
):
    c = lax.axis_index("c")
    s = lax.axis_index("s")
    wid = s * NC + c
    base = wid * SEQ_PER_W

    pltpu.sync_copy(pe_hbm, pe_v)
    pltpu.sync_copy(idx_hbm.at[pl.ds(base, SEQ_PER_W)], idx_v)

    def gather_start(i, rows, sem):
        pltpu.async_copy(
            table_hbm.at[idx_v.at[i, pl.ds(0, 128)]], rows.at[pl.ds(0, 128)], sem
        )
        pltpu.async_copy(
            table_hbm.at[idx_v.at[i, pl.ds(128, 72)]], rows.at[pl.ds(128, 72)], sem
        )

    def gather_wait(rows, sem):
        pltpu.make_async_copy(
            table_hbm.at[idx_v.at[0, pl.ds(0, 128)]], rows.at[pl.ds(0, 128)], sem
        ).wait()
        pltpu.make_async_copy(
            table_hbm.at[idx_v.at[0, pl.ds(128, 72)]], rows.at[pl.ds(128, 72)], sem
        ).wait()

    def add_pe(rows):
        def row_body(r, carry2):
            for j in range(NUM_HID // L):
                sl = pl.ds(j * L, L)
                plsc.addupdate(rows.at[r, sl], pe_v[r, sl])
            return carry2

        lax.fori_loop(0, SEQ_LEN, row_body, 0, unroll=2)

    gather_start(0, rows0, sem0)

    def outer(g, carry):
        s0 = 2 * g
        gather_start(s0 + 1, rows1, sem1)
        gather_wait(rows0, sem0)
        add_pe(rows0)
        pltpu.sync_copy(rows0, out_hbm.at[base + s0])

        @pl.when(s0 + 2 < SEQ_PER_W)
        def _():
            gather_start(s0 + 2, rows0, sem0)

        gather_wait(rows1, sem1)
        add_pe(rows1)
        pltpu.sync_copy(rows1, out_hbm.at[base + s0 + 1])
        return carry

    lax.fori_loop(0, SEQ_PER_W // 2, outer, 0)


@jax.jit
def kernel(input, table):
    pe = _pos_encode(SEQ_LEN, NUM_HID)
    mesh = plsc.VectorSubcoreMesh(core_axis_name="c", subcore_axis_name="s")
    f = pl.kernel(
        _sc_body,
        out_type=jax.ShapeDtypeStruct((BATCH, SEQ_LEN, NUM_HID), jnp.float32),
        mesh=mesh,
        scratch_types=[
            pltpu.VMEM((SEQ_PER_W, SEQ_LEN), jnp.int32),
            pltpu.VMEM((SEQ_LEN, NUM_HID), jnp.float32),
            pltpu.VMEM((SEQ_LEN, NUM_HID), jnp.float32),
            pltpu.VMEM((SEQ_LEN, NUM_HID), jnp.float32),
            pltpu.SemaphoreType.DMA,
            pltpu.SemaphoreType.DMA,
        ],
    )
    return f(input.astype(jnp.int32), table, pe)


# speedup vs baseline: 7.6042x; 1.0031x over previous
import functools

import jax
import jax.numpy as jnp
from jax import lax
from jax.experimental import pallas as pl
from jax.experimental.pallas import tpu as pltpu
from jax.experimental.pallas import tpu_sc as plsc

SEQ_LEN = 200
NUM_HID = 128
BATCH = 4096

_info = plsc.get_sparse_core_info()
NC, NS, L = _info.num_cores, _info.num_subcores, _info.num_lanes
NW = NC * NS
SEQ_PER_W = BATCH // NW


def _pos_encode(seq_len, dim):
    range_even = jnp.arange(dim, dtype=jnp.float32)
    range_even = (range_even / 2.0).astype(jnp.int32).astype(jnp.float32)
    power = range_even / float(dim)
    denom = jnp.power(10000.0, power).reshape(1, dim)
    pos = jnp.arange(seq_len, dtype=jnp.float32).reshape(seq_len, 1)
    arg = pos / denom
    cos_mask = (jnp.arange(seq_len) % 2).astype(bool).reshape(seq_len, 1)
    sin = jnp.where(jnp.logical_not(cos_mask), jnp.sin(arg), 0.0)
    cos = jnp.where(cos_mask, jnp.cos(arg), 0.0)
    return sin + cos


def _sc_body(
    idx_hbm, table_hbm, pe_hbm, out_hbm,
    idx_v, rows0, rows1, pe_v, g0, g1, o0, o1,
):
    c = lax.axis_index("c")
    s = lax.axis_index("s")
    wid = s * NC + c
    base = wid * SEQ_PER_W

    pltpu.sync_copy(pe_hbm, pe_v)
    pltpu.sync_copy(idx_hbm.at[pl.ds(base, SEQ_PER_W)], idx_v)

    def gather_start(i, rows, sem):
        pltpu.async_copy(
            table_hbm.at[idx_v.at[i, pl.ds(0, 128)]], rows.at[pl.ds(0, 128)], sem
        )
        pltpu.async_copy(
            table_hbm.at[idx_v.at[i, pl.ds(128, 72)]], rows.at[pl.ds(128, 72)], sem
        )

    def gather_wait(rows, sem):
        pltpu.make_async_copy(
            table_hbm.at[idx_v.at[0, pl.ds(0, 128)]], rows.at[pl.ds(0, 128)], sem
        ).wait()
        pltpu.make_async_copy(
            table_hbm.at[idx_v.at[0, pl.ds(128, 72)]], rows.at[pl.ds(128, 72)], sem
        ).wait()

    def out_start(rows, i, sem):
        pltpu.async_copy(rows, out_hbm.at[base + i], sem)

    def out_wait(rows, sem):
        pltpu.make_async_copy(rows, out_hbm.at[base], sem).wait()

    def add_pe(rows):
        def row_body(r, carry2):
            for j in range(NUM_HID // L):
                sl = pl.ds(j * L, L)
                plsc.addupdate(rows.at[r, sl], pe_v[r, sl])
            return carry2

        lax.fori_loop(0, SEQ_LEN, row_body, 0, unroll=4)

    gather_start(0, rows0, g0)

    def outer(g, carry):
        s0 = 2 * g

        @pl.when(g > 0)
        def _():
            out_wait(rows1, o1)

        gather_start(s0 + 1, rows1, g1)
        gather_wait(rows0, g0)
        add_pe(rows0)
        out_start(rows0, s0, o0)

        @pl.when(s0 + 2 < SEQ_PER_W)
        def _():
            out_wait(rows0, o0)
            gather_start(s0 + 2, rows0, g0)

        gather_wait(rows1, g1)
        add_pe(rows1)
        out_start(rows1, s0 + 1, o1)
        return carry

    lax.fori_loop(0, SEQ_PER_W // 2, outer, 0)
    out_wait(rows0, o0)
    out_wait(rows1, o1)


@jax.jit
def kernel(input, table):
    pe = _pos_encode(SEQ_LEN, NUM_HID)
    mesh = plsc.VectorSubcoreMesh(core_axis_name="c", subcore_axis_name="s")
    f = pl.kernel(
        _sc_body,
        out_type=jax.ShapeDtypeStruct((BATCH, SEQ_LEN, NUM_HID), jnp.float32),
        mesh=mesh,
        scratch_types=[
            pltpu.VMEM((SEQ_PER_W, SEQ_LEN), jnp.int32),
            pltpu.VMEM((SEQ_LEN, NUM_HID), jnp.float32),
            pltpu.VMEM((SEQ_LEN, NUM_HID), jnp.float32),
            pltpu.VMEM((SEQ_LEN, NUM_HID), jnp.float32),
            pltpu.SemaphoreType.DMA,
            pltpu.SemaphoreType.DMA,
            pltpu.SemaphoreType.DMA,
            pltpu.SemaphoreType.DMA,
        ],
    )
    return f(input.astype(jnp.int32), table, pe)


# speedup vs baseline: 9.1526x; 1.2036x over previous
import functools

import jax
import jax.numpy as jnp
from jax import lax
from jax.experimental import pallas as pl
from jax.experimental.pallas import tpu as pltpu
from jax.experimental.pallas import tpu_sc as plsc

SEQ_LEN = 200
NUM_HID = 128
BATCH = 4096

_info = plsc.get_sparse_core_info()
NC, NS, L = _info.num_cores, _info.num_subcores, _info.num_lanes
NW = NC * NS
SEQ_PER_W = BATCH // NW


def _pos_encode(seq_len, dim):
    range_even = jnp.arange(dim, dtype=jnp.float32)
    range_even = (range_even / 2.0).astype(jnp.int32).astype(jnp.float32)
    power = range_even / float(dim)
    denom = jnp.power(10000.0, power).reshape(1, dim)
    pos = jnp.arange(seq_len, dtype=jnp.float32).reshape(seq_len, 1)
    arg = pos / denom
    cos_mask = (jnp.arange(seq_len) % 2).astype(bool).reshape(seq_len, 1)
    sin = jnp.where(jnp.logical_not(cos_mask), jnp.sin(arg), 0.0)
    cos = jnp.where(cos_mask, jnp.cos(arg), 0.0)
    return sin + cos


def _sc_body(
    idx_hbm, table_hbm, pe_hbm, out_hbm,
    idx0, idx1, idx2, rows0, rows1, rows2, pe_v,
    g0, g1, g2, o0, o1, o2, i0, i1, i2,
):
    c = lax.axis_index("c")
    s = lax.axis_index("s")
    wid = s * NC + c
    base = wid * SEQ_PER_W

    idxs = (idx0, idx1, idx2)
    rows = (rows0, rows1, rows2)
    gsem = (g0, g1, g2)
    osem = (o0, o1, o2)
    isem = (i0, i1, i2)

    pltpu.sync_copy(pe_hbm, pe_v)
    for k in range(3):
        pltpu.sync_copy(idx_hbm.at[base + k], idxs[k])

    def idx_start(t, k):
        pltpu.async_copy(idx_hbm.at[base + t], idxs[k], isem[k])

    def idx_wait(k):
        pltpu.make_async_copy(idx_hbm.at[base], idxs[k], isem[k]).wait()

    def gather_start(t, k):
        pltpu.async_copy(
            table_hbm.at[idxs[k].at[pl.ds(0, 128)]], rows[k].at[pl.ds(0, 128)], gsem[k]
        )
        pltpu.async_copy(
            table_hbm.at[idxs[k].at[pl.ds(128, 72)]], rows[k].at[pl.ds(128, 72)], gsem[k]
        )

    def gather_wait(k):
        pltpu.make_async_copy(
            table_hbm.at[idxs[k].at[pl.ds(0, 128)]], rows[k].at[pl.ds(0, 128)], gsem[k]
        ).wait()
        pltpu.make_async_copy(
            table_hbm.at[idxs[k].at[pl.ds(128, 72)]], rows[k].at[pl.ds(128, 72)], gsem[k]
        ).wait()

    def out_start(t, k):
        pltpu.async_copy(rows[k], out_hbm.at[base + t], osem[k])

    def out_wait(k):
        pltpu.make_async_copy(rows[k], out_hbm.at[base], osem[k]).wait()

    def add_pe(k):
        def row_body(r, carry2):
            for j in range(NUM_HID // L):
                sl = pl.ds(j * L, L)
                plsc.addupdate(rows[k].at[r, sl], pe_v[r, sl])
            return carry2

        lax.fori_loop(0, SEQ_LEN, row_body, 0, unroll=4)

    gather_start(0, 0)
    gather_start(1, 1)

    def outer(g, carry):
        t0 = 3 * g
        for k in range(3):
            t = t0 + k
            gather_wait(k)

            @pl.when(t < SEQ_PER_W - 3)
            def _():
                idx_start(t + 3, k)

            add_pe(k)
            out_start(t, k)
            kk = (k + 2) % 3

            if k == 0:
                @pl.when(t >= 1)
                def _():
                    out_wait(kk)
                    idx_wait(kk)
            else:
                out_wait(kk)
                idx_wait(kk)
            gather_start(t + 2, kk)
        return carry

    lax.fori_loop(0, (SEQ_PER_W - 2) // 3, outer, 0)

    for k, t in ((0, SEQ_PER_W - 2), (1, SEQ_PER_W - 1)):
        gather_wait(k)
        add_pe(k)
        out_start(t, k)
    out_wait(2)
    out_wait(0)
    out_wait(1)


@jax.jit
def kernel(input, table):
    pe = _pos_encode(SEQ_LEN, NUM_HID)
    mesh = plsc.VectorSubcoreMesh(core_axis_name="c", subcore_axis_name="s")
    f = pl.kernel(
        _sc_body,
        out_type=jax.ShapeDtypeStruct((BATCH, SEQ_LEN, NUM_HID), jnp.float32),
        mesh=mesh,
        scratch_types=[
            pltpu.VMEM((SEQ_LEN,), jnp.int32),
            pltpu.VMEM((SEQ_LEN,), jnp.int32),
            pltpu.VMEM((SEQ_LEN,), jnp.int32),
            pltpu.VMEM((SEQ_LEN, NUM_HID), jnp.float32),
            pltpu.VMEM((SEQ_LEN, NUM_HID), jnp.float32),
            pltpu.VMEM((SEQ_LEN, NUM_HID), jnp.float32),
            pltpu.VMEM((SEQ_LEN, NUM_HID), jnp.float32),
        ] + [pltpu.SemaphoreType.DMA] * 9,
    )
    return f(input.astype(jnp.int32), table, pe)


# speedup vs baseline: 9.1557x; 1.0003x over previous
import functools

import jax
import jax.numpy as jnp
from jax import lax
from jax.experimental import pallas as pl
from jax.experimental.pallas import tpu as pltpu
from jax.experimental.pallas import tpu_sc as plsc

SEQ_LEN = 200
NUM_HID = 128
BATCH = 4096

_info = plsc.get_sparse_core_info()
NC, NS, L = _info.num_cores, _info.num_subcores, _info.num_lanes
NW = NC * NS
SEQ_PER_W = BATCH // NW


def _pos_encode(seq_len, dim):
    range_even = jnp.arange(dim, dtype=jnp.float32)
    range_even = (range_even / 2.0).astype(jnp.int32).astype(jnp.float32)
    power = range_even / float(dim)
    denom = jnp.power(10000.0, power).reshape(1, dim)
    pos = jnp.arange(seq_len, dtype=jnp.float32).reshape(seq_len, 1)
    arg = pos / denom
    cos_mask = (jnp.arange(seq_len) % 2).astype(bool).reshape(seq_len, 1)
    sin = jnp.where(jnp.logical_not(cos_mask), jnp.sin(arg), 0.0)
    cos = jnp.where(cos_mask, jnp.cos(arg), 0.0)
    return sin + cos


def _sc_body(
    idx_hbm, table_hbm, pe_hbm, out_hbm,
    idx0, idx1, idx2, rows0, rows1, rows2, pe_v,
    g0, g1, g2, o0, o1, o2, i0, i1, i2,
):
    c = lax.axis_index("c")
    s = lax.axis_index("s")
    wid = s * NC + c
    base = wid * SEQ_PER_W

    idxs = (idx0, idx1, idx2)
    rows = (rows0, rows1, rows2)
    gsem = (g0, g1, g2)
    osem = (o0, o1, o2)
    isem = (i0, i1, i2)

    pltpu.sync_copy(pe_hbm, pe_v)
    for k in range(3):
        pltpu.sync_copy(idx_hbm.at[base + k], idxs[k])

    def idx_start(t, k):
        pltpu.async_copy(idx_hbm.at[base + t], idxs[k], isem[k])

    def idx_wait(k):
        pltpu.make_async_copy(idx_hbm.at[base], idxs[k], isem[k]).wait()

    def gather_start(t, k):
        pltpu.async_copy(table_hbm.at[idxs[k]], rows[k], gsem[k])

    def gather_wait(k):
        pltpu.make_async_copy(table_hbm.at[idxs[k]], rows[k], gsem[k]).wait()

    def out_start(t, k):
        pltpu.async_copy(rows[k], out_hbm.at[base + t], osem[k])

    def out_wait(k):
        pltpu.make_async_copy(rows[k], out_hbm.at[base], osem[k]).wait()

    def add_pe(k):
        def row_body(r, carry2):
            for j in range(NUM_HID // L):
                sl = pl.ds(j * L, L)
                plsc.addupdate(rows[k].at[r, sl], pe_v[r, sl])
            return carry2

        lax.fori_loop(0, SEQ_LEN, row_body, 0, unroll=4)

    gather_start(0, 0)
    gather_start(1, 1)

    def outer(g, carry):
        t0 = 3 * g
        for k in range(3):
            t = t0 + k
            gather_wait(k)

            @pl.when(t < SEQ_PER_W - 3)
            def _():
                idx_start(t + 3, k)

            add_pe(k)
            out_start(t, k)
            kk = (k + 2) % 3

            if k == 0:
                @pl.when(t >= 1)
                def _():
                    out_wait(kk)
                    idx_wait(kk)
            else:
                out_wait(kk)
                idx_wait(kk)
            gather_start(t + 2, kk)
        return carry

    lax.fori_loop(0, (SEQ_PER_W - 2) // 3, outer, 0)

    for k, t in ((0, SEQ_PER_W - 2), (1, SEQ_PER_W - 1)):
        gather_wait(k)
        add_pe(k)
        out_start(t, k)
    out_wait(2)
    out_wait(0)
    out_wait(1)


@jax.jit
def kernel(input, table):
    pe = _pos_encode(SEQ_LEN, NUM_HID)
    mesh = plsc.VectorSubcoreMesh(core_axis_name="c", subcore_axis_name="s")
    f = pl.kernel(
        _sc_body,
        out_type=jax.ShapeDtypeStruct((BATCH, SEQ_LEN, NUM_HID), jnp.float32),
        mesh=mesh,
        scratch_types=[
            pltpu.VMEM((SEQ_LEN,), jnp.int32),
            pltpu.VMEM((SEQ_LEN,), jnp.int32),
            pltpu.VMEM((SEQ_LEN,), jnp.int32),
            pltpu.VMEM((SEQ_LEN, NUM_HID), jnp.float32),
            pltpu.VMEM((SEQ_LEN, NUM_HID), jnp.float32),
            pltpu.VMEM((SEQ_LEN, NUM_HID), jnp.float32),
            pltpu.VMEM((SEQ_LEN, NUM_HID), jnp.float32),
        ] + [pltpu.SemaphoreType.DMA] * 9,
    )
    return f(input.astype(jnp.int32), table, pe)
